# Initial kernel scaffold; baseline (speedup 1.0000x reference)
import jax
import jax.numpy as jnp
from jax.experimental import pallas as pl


def kernel(x, W):
    raise NotImplementedError("write your pallas kernel here")


# speedup vs baseline: 2.1060x; 2.1060x over previous
import functools

import jax
import jax.numpy as jnp
from jax.experimental import pallas as pl

NUM_TOKENS = 16384
HIDDEN = 2048
NUM_EXPERTS = 64
BLOCK_TOKENS = 512


def _router_kernel(x_ref, wt_ref, vals_ref, idx_ref):
    scores = jnp.dot(x_ref[...], wt_ref[...],
                     preferred_element_type=jnp.float32)
    iota = jax.lax.broadcasted_iota(jnp.int32, scores.shape, 1)
    m1 = jnp.max(scores, axis=1, keepdims=True)
    i1 = jnp.min(jnp.where(scores == m1, iota, NUM_EXPERTS),
                 axis=1, keepdims=True)
    masked = jnp.where(iota == i1, -jnp.inf, scores)
    m2 = jnp.max(masked, axis=1, keepdims=True)
    i2 = jnp.min(jnp.where(masked == m2, iota, NUM_EXPERTS),
                 axis=1, keepdims=True)
    vals_ref[...] = jnp.concatenate([m1, m2], axis=1)
    idx_ref[...] = jnp.concatenate([i1, i2], axis=1)


@jax.jit
def kernel(x, W):
    grid = (NUM_TOKENS // BLOCK_TOKENS,)
    vals, idx = pl.pallas_call(
        _router_kernel,
        grid=grid,
        in_specs=[
            pl.BlockSpec((BLOCK_TOKENS, HIDDEN), lambda i: (i, 0)),
            pl.BlockSpec((HIDDEN, NUM_EXPERTS), lambda i: (0, 0)),
        ],
        out_specs=[
            pl.BlockSpec((BLOCK_TOKENS, 2), lambda i: (i, 0)),
            pl.BlockSpec((BLOCK_TOKENS, 2), lambda i: (i, 0)),
        ],
        out_shape=[
            jax.ShapeDtypeStruct((NUM_TOKENS, 2), jnp.float32),
            jax.ShapeDtypeStruct((NUM_TOKENS, 2), jnp.int32),
        ],
    )(x, W.T)
    return vals, idx


# speedup vs baseline: 2.4636x; 1.1698x over previous
import functools

import jax
import jax.numpy as jnp
from jax.experimental import pallas as pl

NUM_TOKENS = 16384
HIDDEN = 2048
NUM_EXPERTS = 64
BLOCK_TOKENS = 1024


def _router_kernel(x_ref, wt_ref, vals_ref, idx_ref):
    scores = jnp.dot(x_ref[...], wt_ref[...],
                     preferred_element_type=jnp.float32)
    iota = jax.lax.broadcasted_iota(jnp.int32, scores.shape, 1)
    m1 = jnp.max(scores, axis=1, keepdims=True)
    i1 = jnp.min(jnp.where(scores == m1, iota, NUM_EXPERTS),
                 axis=1, keepdims=True)
    masked = jnp.where(iota == i1, -jnp.inf, scores)
    m2 = jnp.max(masked, axis=1, keepdims=True)
    i2 = jnp.min(jnp.where(masked == m2, iota, NUM_EXPERTS),
                 axis=1, keepdims=True)
    vals_ref[...] = jnp.concatenate([m1, m2], axis=1)
    idx_ref[...] = jnp.concatenate([i1, i2], axis=1)


@jax.jit
def kernel(x, W):
    grid = (NUM_TOKENS // BLOCK_TOKENS,)
    vals, idx = pl.pallas_call(
        _router_kernel,
        grid=grid,
        in_specs=[
            pl.BlockSpec((BLOCK_TOKENS, HIDDEN), lambda i: (i, 0)),
            pl.BlockSpec((HIDDEN, NUM_EXPERTS), lambda i: (0, 0)),
        ],
        out_specs=[
            pl.BlockSpec((BLOCK_TOKENS, 2), lambda i: (i, 0)),
            pl.BlockSpec((BLOCK_TOKENS, 2), lambda i: (i, 0)),
        ],
        out_shape=[
            jax.ShapeDtypeStruct((NUM_TOKENS, 2), jnp.float32),
            jax.ShapeDtypeStruct((NUM_TOKENS, 2), jnp.int32),
        ],
    )(x, W.T)
    return vals, idx


# speedup vs baseline: 2.5591x; 1.0387x over previous
import functools

import jax
import jax.numpy as jnp
from jax.experimental import pallas as pl

NUM_TOKENS = 16384
HIDDEN = 2048
NUM_EXPERTS = 64
BLOCK_TOKENS = 2048


def _router_kernel(x_ref, wt_ref, vals_ref, idx_ref):
    scores = jnp.dot(x_ref[...], wt_ref[...],
                     preferred_element_type=jnp.float32)
    iota = jax.lax.broadcasted_iota(jnp.int32, scores.shape, 1)
    m1 = jnp.max(scores, axis=1, keepdims=True)
    i1 = jnp.min(jnp.where(scores == m1, iota, NUM_EXPERTS),
                 axis=1, keepdims=True)
    masked = jnp.where(iota == i1, -jnp.inf, scores)
    m2 = jnp.max(masked, axis=1, keepdims=True)
    i2 = jnp.min(jnp.where(masked == m2, iota, NUM_EXPERTS),
                 axis=1, keepdims=True)
    vals_ref[...] = jnp.concatenate([m1, m2], axis=1)
    idx_ref[...] = jnp.concatenate([i1, i2], axis=1)


@jax.jit
def kernel(x, W):
    grid = (NUM_TOKENS // BLOCK_TOKENS,)
    vals, idx = pl.pallas_call(
        _router_kernel,
        grid=grid,
        in_specs=[
            pl.BlockSpec((BLOCK_TOKENS, HIDDEN), lambda i: (i, 0)),
            pl.BlockSpec((HIDDEN, NUM_EXPERTS), lambda i: (0, 0)),
        ],
        out_specs=[
            pl.BlockSpec((BLOCK_TOKENS, 2), lambda i: (i, 0)),
            pl.BlockSpec((BLOCK_TOKENS, 2), lambda i: (i, 0)),
        ],
        out_shape=[
            jax.ShapeDtypeStruct((NUM_TOKENS, 2), jnp.float32),
            jax.ShapeDtypeStruct((NUM_TOKENS, 2), jnp.int32),
        ],
    )(x, W.T)
    return vals, idx
